# Initial kernel scaffold; baseline (speedup 1.0000x reference)
import jax
import jax.numpy as jnp
from jax.experimental import pallas as pl


def kernel(x, edge_index, Wq0, bq0, Wk0, bk0, W0, b0, Wq1, bq1, Wk1, bk1, W1, b1):
    raise NotImplementedError("write your pallas kernel here")


# speedup vs baseline: 63.3931x; 63.3931x over previous
import functools

import jax
import jax.numpy as jnp
from jax import lax
from jax.experimental import pallas as pl
from jax.experimental.pallas import tpu as pltpu
from jax.experimental.pallas import tpu_sc as plsc

N = 10000
E = 160000
NP = 10240
EP = 172032
W_WIN = 128
NC = 2
NS = 16
ROWS_PER_TILE = NP // NS


def _edge_sc_kernel(width):
    per_tile = EP // (NC * NS)
    n_win = per_tile // W_WIN
    mesh = plsc.VectorSubcoreMesh(core_axis_name="c", subcore_axis_name="s")

    @functools.partial(
        pl.kernel,
        mesh=mesh,
        compiler_params=pltpu.CompilerParams(use_tc_tiling_on_sc=False),
        out_type=jax.ShapeDtypeStruct((NC, NP, width), jnp.float32),
        scratch_types=[
            pltpu.VMEM((W_WIN,), jnp.int32),
            pltpu.VMEM((W_WIN,), jnp.int32),
            pltpu.VMEM((W_WIN, 16), jnp.float32),
            pltpu.VMEM((W_WIN, width), jnp.float32),
            pltpu.VMEM((W_WIN, width), jnp.float32),
            pltpu.VMEM_SHARED((NP, width), jnp.float32),
            pltpu.SemaphoreType.DMA,
            pltpu.SemaphoreType.DMA,
        ],
    )
    def k(row_h, col_h, tq_h, tkv_h, out_h, idx_r, idx_c, qr, kvr, upd, acc,
          sem1, sem2):
        cid = lax.axis_index("c")
        sid = lax.axis_index("s")
        wid = cid * NS + sid
        r0 = sid * ROWS_PER_TILE

        zero16 = jnp.zeros((16,), jnp.float32)

        def zr(i, c):
            for j in range(width // 16):
                upd[i, pl.ds(16 * j, 16)] = zero16
            return c

        lax.fori_loop(0, W_WIN, zr, 0)
        for j in range(ROWS_PER_TILE // W_WIN):
            pltpu.sync_copy(upd, acc.at[pl.ds(r0 + j * W_WIN, W_WIN)])
        plsc.subcore_barrier()

        base = wid * per_tile
        lane = lax.iota(jnp.int32, 16)
        zidx = jnp.zeros((16,), jnp.int32)

        def win(w, carry):
            off = base + w * W_WIN
            pltpu.sync_copy(row_h.at[pl.ds(off, W_WIN)], idx_r)
            pltpu.sync_copy(col_h.at[pl.ds(off, W_WIN)], idx_c)
            cp1 = pltpu.async_copy(tq_h.at[idx_r], qr, sem1)
            cp2 = pltpu.async_copy(tkv_h.at[idx_c], kvr, sem2)
            cp1.wait()
            cp2.wait()

            def edge(e, c2):
                q = qr[e]
                if width == 80:
                    kv0 = kvr[e, pl.ds(0, 16)]
                    ex = jnp.exp(q * kv0)
                    upd[e, pl.ds(0, 16)] = ex
                    for j in range(4):
                        bj = _take16(ex, (lane >> 3) + 2 * j)
                        vj = kvr[e, pl.ds(16 + 16 * j, 16)]
                        upd[e, pl.ds(16 + 16 * j, 16)] = bj * vj
                else:
                    kv = kvr[e]
                    ex = jnp.exp(q * kv)
                    b0v = _take16(ex, zidx)
                    upd[e] = b0v * kv
                return c2

            lax.fori_loop(0, W_WIN, edge, 0)
            pltpu.sync_copy(upd, acc.at[idx_r], add=True)
            return carry

        lax.fori_loop(0, n_win, win, 0)
        plsc.subcore_barrier()

        for j in range(ROWS_PER_TILE // W_WIN):
            sl = pl.ds(r0 + j * W_WIN, W_WIN)
            pltpu.sync_copy(acc.at[sl], upd)
            pltpu.sync_copy(upd, out_h.at[cid, sl])

    return k


def _take16(x, idx):
    return lax.gather(
        x, idx[:, None],
        dimension_numbers=lax.GatherDimensionNumbers(
            offset_dims=(), collapsed_slice_dims=(0,), start_index_map=(0,)),
        slice_sizes=(1,),
        mode=lax.GatherScatterMode.PROMISE_IN_BOUNDS)


_edge_sc_80 = _edge_sc_kernel(80)
_edge_sc_16 = _edge_sc_kernel(16)

_HIGHEST = lax.Precision.HIGHEST


def _prep_kernel(x_ref, wq_ref, bq_ref, wkv_ref, bkv_ref, tq_ref, tkv_ref):
    x = x_ref[...]
    tq_ref[...] = jax.nn.relu(
        jnp.dot(x, wq_ref[...], preferred_element_type=jnp.float32,
                precision=_HIGHEST) + bq_ref[...])
    t = jnp.dot(x, wkv_ref[...], preferred_element_type=jnp.float32,
                precision=_HIGHEST) + bkv_ref[...]
    ci = lax.broadcasted_iota(jnp.int32, t.shape, 1)
    tkv_ref[...] = jnp.where(ci < 16, jax.nn.relu(t), t)


def _mid_kernel(a_ref, b0_ref, wq1_ref, bq1_ref, wkv1_ref, bkv1_ref,
                tq1_ref, tkv1_ref):
    s = a_ref[0] + a_ref[1]
    d8 = s[:, 0:8]
    r = lax.broadcasted_iota(jnp.int32, (8, 64), 0)
    c = lax.broadcasted_iota(jnp.int32, (8, 64), 1)
    onehot = (r == c // 8).astype(jnp.float32)
    d64 = jnp.dot(d8, onehot, preferred_element_type=jnp.float32,
                  precision=_HIGHEST)
    h = jax.nn.relu(s[:, 16:80] / (d64 + 1e-16) + b0_ref[...])
    tq1_ref[...] = jax.nn.relu(
        jnp.dot(h, wq1_ref[...], preferred_element_type=jnp.float32,
                precision=_HIGHEST) + bq1_ref[...])
    t = jnp.dot(h, wkv1_ref[...], preferred_element_type=jnp.float32,
                precision=_HIGHEST) + bkv1_ref[...]
    ci = lax.broadcasted_iota(jnp.int32, t.shape, 1)
    tkv1_ref[...] = jnp.where(ci < 8, jax.nn.relu(t), t)


def _final_kernel(a_ref, b1_ref, out_ref):
    s = a_ref[0] + a_ref[1]
    denom = s[:, 7:8]
    msg = s[:, 8:15]
    res = msg / (denom + 1e-16)
    z = jnp.zeros((res.shape[0], 1), jnp.float32)
    out_ref[...] = jnp.concatenate([res, z], axis=1) + b1_ref[...]


def kernel(x, edge_index, Wq0, bq0, Wk0, bk0, W0, b0, Wq1, bq1, Wk1, bk1,
           W1, b1):
    f32 = jnp.float32
    xp = jnp.pad(x, ((0, NP - N), (0, 0)))
    row = edge_index[0]
    col = edge_index[1]
    loops = jnp.arange(N, dtype=jnp.int32)
    padi = 10000 + (jnp.arange(EP - E - N, dtype=jnp.int32) % 64)
    rowp = jnp.concatenate([row, loops, padi])
    colp = jnp.concatenate([col, loops, padi])

    wq16 = jnp.concatenate([Wq0, jnp.zeros((256, 8), f32)], 1)
    bq16 = jnp.concatenate([bq0, jnp.zeros((8,), f32)]).reshape(1, 16)
    wkv80 = jnp.concatenate([Wk0, jnp.zeros((256, 8), f32), W0], 1)
    bkv80 = jnp.concatenate([bk0, jnp.zeros((72,), f32)]).reshape(1, 80)

    wq1p = jnp.concatenate([Wq1, jnp.zeros((64, 15), f32)], 1)
    bq1p = jnp.concatenate([bq1, jnp.zeros((15,), f32)]).reshape(1, 16)
    wkv1p = jnp.concatenate(
        [Wk1, jnp.zeros((64, 7), f32), W1, jnp.zeros((64, 1), f32)], 1)
    bkv1p = jnp.concatenate(
        [bk1, jnp.zeros((6,), f32), jnp.ones((1,), f32),
         jnp.zeros((8,), f32)]).reshape(1, 16)
    b1p = jnp.concatenate([b1, jnp.zeros((1,), f32)]).reshape(1, 8)
    b0r = b0.reshape(1, 64)

    blk = 1024
    grid = NP // blk

    tq0, tkv0 = pl.pallas_call(
        _prep_kernel,
        grid=(grid,),
        in_specs=[
            pl.BlockSpec((blk, 256), lambda i: (i, 0)),
            pl.BlockSpec((256, 16), lambda i: (0, 0)),
            pl.BlockSpec((1, 16), lambda i: (0, 0)),
            pl.BlockSpec((256, 80), lambda i: (0, 0)),
            pl.BlockSpec((1, 80), lambda i: (0, 0)),
        ],
        out_specs=[
            pl.BlockSpec((blk, 16), lambda i: (i, 0)),
            pl.BlockSpec((blk, 80), lambda i: (i, 0)),
        ],
        out_shape=[
            jax.ShapeDtypeStruct((NP, 16), f32),
            jax.ShapeDtypeStruct((NP, 80), f32),
        ],
    )(xp, wq16, bq16, wkv80, bkv80)

    acc0 = _edge_sc_80(rowp, colp, tq0, tkv0)

    tq1, tkv1 = pl.pallas_call(
        _mid_kernel,
        grid=(grid,),
        in_specs=[
            pl.BlockSpec((2, blk, 80), lambda i: (0, i, 0)),
            pl.BlockSpec((1, 64), lambda i: (0, 0)),
            pl.BlockSpec((64, 16), lambda i: (0, 0)),
            pl.BlockSpec((1, 16), lambda i: (0, 0)),
            pl.BlockSpec((64, 16), lambda i: (0, 0)),
            pl.BlockSpec((1, 16), lambda i: (0, 0)),
        ],
        out_specs=[
            pl.BlockSpec((blk, 16), lambda i: (i, 0)),
            pl.BlockSpec((blk, 16), lambda i: (i, 0)),
        ],
        out_shape=[
            jax.ShapeDtypeStruct((NP, 16), f32),
            jax.ShapeDtypeStruct((NP, 16), f32),
        ],
    )(acc0, b0r, wq1p, bq1p, wkv1p, bkv1p)

    acc1 = _edge_sc_16(rowp, colp, tq1, tkv1)

    outp = pl.pallas_call(
        _final_kernel,
        grid=(grid,),
        in_specs=[
            pl.BlockSpec((2, blk, 16), lambda i: (0, i, 0)),
            pl.BlockSpec((1, 8), lambda i: (0, 0)),
        ],
        out_specs=pl.BlockSpec((blk, 8), lambda i: (i, 0)),
        out_shape=jax.ShapeDtypeStruct((NP, 8), f32),
    )(acc1, b1p)

    return outp[:N, :7]


# speedup vs baseline: 100.0464x; 1.5782x over previous
import functools

import jax
import jax.numpy as jnp
from jax import lax
from jax.experimental import pallas as pl
from jax.experimental.pallas import tpu as pltpu
from jax.experimental.pallas import tpu_sc as plsc

N = 10000
E = 160000
NP = 10240
EP = 172032
W_WIN = 128
NC = 2
NS = 16
ROWS_PER_TILE = NP // NS


def _edge_sc_kernel(width):
    per_tile = EP // (NC * NS)
    n_win = per_tile // W_WIN
    mesh = plsc.VectorSubcoreMesh(core_axis_name="c", subcore_axis_name="s")

    @functools.partial(
        pl.kernel,
        mesh=mesh,
        compiler_params=pltpu.CompilerParams(use_tc_tiling_on_sc=False),
        out_type=jax.ShapeDtypeStruct((NC, NP, width), jnp.float32),
        scratch_types=[
            pltpu.VMEM((2, 2, W_WIN), jnp.int32),
            pltpu.VMEM((2, W_WIN), jnp.int32),
            pltpu.VMEM((2, W_WIN, 16), jnp.float32),
            pltpu.VMEM((2, W_WIN, width), jnp.float32),
            pltpu.VMEM((2, W_WIN, width), jnp.float32),
            pltpu.VMEM_SHARED((NP, width), jnp.float32),
            pltpu.SemaphoreType.DMA((2,)),
            pltpu.SemaphoreType.DMA((2,)),
            pltpu.SemaphoreType.DMA((2,)),
            pltpu.SemaphoreType.DMA((2,)),
        ],
    )
    def k(idx_h, tq_h, tkv_h, out_h, idxb, sidx, qr, kvr, upd, acc,
          sem_i, sem_q, sem_kv, sem_s):
        cid = lax.axis_index("c")
        sid = lax.axis_index("s")
        wid = cid * NS + sid
        r0 = sid * ROWS_PER_TILE

        zero16 = jnp.zeros((16,), jnp.float32)

        def zr(i, c):
            for j in range(width // 16):
                upd[0, i, pl.ds(16 * j, 16)] = zero16
            return c

        lax.fori_loop(0, W_WIN, zr, 0)
        for j in range(ROWS_PER_TILE // W_WIN):
            pltpu.sync_copy(upd.at[0], acc.at[pl.ds(r0 + j * W_WIN, W_WIN)])
        plsc.subcore_barrier()

        base = wid * n_win
        lane = lax.iota(jnp.int32, 16)
        zidx = jnp.zeros((16,), jnp.int32)

        def start_idx(w, p):
            pltpu.async_copy(idx_h.at[base + w], idxb.at[p], sem_i.at[p])

        def start_gathers(w, p):
            pltpu.async_copy(tq_h.at[idxb.at[p, 0]], qr.at[p], sem_q.at[p])
            pltpu.async_copy(tkv_h.at[idxb.at[p, 1]], kvr.at[p], sem_kv.at[p])

        def wait_idx(p):
            pltpu.make_async_copy(idx_h.at[base], idxb.at[p],
                                  sem_i.at[p]).wait()

        def wait_gathers(p):
            pltpu.make_async_copy(tq_h.at[idxb.at[p, 0]], qr.at[p],
                                  sem_q.at[p]).wait()
            pltpu.make_async_copy(tkv_h.at[idxb.at[p, 1]], kvr.at[p],
                                  sem_kv.at[p]).wait()

        def wait_scatter(p):
            pltpu.make_async_copy(upd.at[p], acc.at[sidx.at[p]],
                                  sem_s.at[p]).wait()

        def compute(p):
            def edge(e, c2):
                for u in range(2):
                    eu = 2 * e + u
                    q = qr[p, eu]
                    if width == 80:
                        kv0 = kvr[p, eu, pl.ds(0, 16)]
                        ex = jnp.exp(q * kv0)
                        upd[p, eu, pl.ds(0, 16)] = ex
                        for j in range(4):
                            bj = _take16(ex, (lane >> 3) + 2 * j)
                            vj = kvr[p, eu, pl.ds(16 + 16 * j, 16)]
                            upd[p, eu, pl.ds(16 + 16 * j, 16)] = bj * vj
                    else:
                        kv = kvr[p, eu]
                        ex = jnp.exp(q * kv)
                        b0v = _take16(ex, zidx)
                        upd[p, eu] = b0v * kv
                return c2

            lax.fori_loop(0, W_WIN // 2, edge, 0)

        start_idx(0, 0)
        start_idx(1, 1)
        wait_idx(0)
        start_gathers(0, 0)

        def body(w, p):
            @pl.when(w >= 2)
            def _():
                wait_scatter(p)

            wait_gathers(p)
            for i in range(W_WIN // 16):
                sidx[p, pl.ds(16 * i, 16)] = idxb[p, 0, pl.ds(16 * i, 16)]

            @pl.when(w + 1 < n_win)
            def _():
                wait_idx(1 - p)
                start_gathers(w + 1, 1 - p)

            @pl.when(w + 2 < n_win)
            def _():
                start_idx(w + 2, p)

            compute(p)
            pltpu.async_copy(upd.at[p], acc.at[sidx.at[p]], sem_s.at[p],
                             add=True)

        def outer(g, carry):
            body(2 * g, 0)
            body(2 * g + 1, 1)
            return carry

        lax.fori_loop(0, n_win // 2, outer, 0)
        wait_scatter(0)
        wait_scatter(1)
        plsc.subcore_barrier()

        for j in range(ROWS_PER_TILE // W_WIN):
            sl = pl.ds(r0 + j * W_WIN, W_WIN)
            pltpu.sync_copy(acc.at[sl], upd.at[0])
            pltpu.sync_copy(upd.at[0], out_h.at[cid, sl])

    return k


def _take16(x, idx):
    return lax.gather(
        x, idx[:, None],
        dimension_numbers=lax.GatherDimensionNumbers(
            offset_dims=(), collapsed_slice_dims=(0,), start_index_map=(0,)),
        slice_sizes=(1,),
        mode=lax.GatherScatterMode.PROMISE_IN_BOUNDS)


_edge_sc_80 = _edge_sc_kernel(80)
_edge_sc_16 = _edge_sc_kernel(16)

_HIGHEST = lax.Precision.HIGHEST


def _prep_kernel(x_ref, wq_ref, bq_ref, wkv_ref, bkv_ref, tq_ref, tkv_ref):
    x = x_ref[...]
    tq_ref[...] = jax.nn.relu(
        jnp.dot(x, wq_ref[...], preferred_element_type=jnp.float32,
                precision=_HIGHEST) + bq_ref[...])
    t = jnp.dot(x, wkv_ref[...], preferred_element_type=jnp.float32,
                precision=_HIGHEST) + bkv_ref[...]
    ci = lax.broadcasted_iota(jnp.int32, t.shape, 1)
    tkv_ref[...] = jnp.where(ci < 16, jax.nn.relu(t), t)


def _mid_kernel(a_ref, b0_ref, wq1_ref, bq1_ref, wkv1_ref, bkv1_ref,
                tq1_ref, tkv1_ref):
    s = a_ref[0] + a_ref[1]
    d8 = s[:, 0:8]
    r = lax.broadcasted_iota(jnp.int32, (8, 64), 0)
    c = lax.broadcasted_iota(jnp.int32, (8, 64), 1)
    onehot = (r == c // 8).astype(jnp.float32)
    d64 = jnp.dot(d8, onehot, preferred_element_type=jnp.float32,
                  precision=_HIGHEST)
    h = jax.nn.relu(s[:, 16:80] / (d64 + 1e-16) + b0_ref[...])
    tq1_ref[...] = jax.nn.relu(
        jnp.dot(h, wq1_ref[...], preferred_element_type=jnp.float32,
                precision=_HIGHEST) + bq1_ref[...])
    t = jnp.dot(h, wkv1_ref[...], preferred_element_type=jnp.float32,
                precision=_HIGHEST) + bkv1_ref[...]
    ci = lax.broadcasted_iota(jnp.int32, t.shape, 1)
    tkv1_ref[...] = jnp.where(ci < 8, jax.nn.relu(t), t)


def _final_kernel(a_ref, b1_ref, out_ref):
    s = a_ref[0] + a_ref[1]
    denom = s[:, 7:8]
    msg = s[:, 8:15]
    res = msg / (denom + 1e-16)
    z = jnp.zeros((res.shape[0], 1), jnp.float32)
    out_ref[...] = jnp.concatenate([res, z], axis=1) + b1_ref[...]


def kernel(x, edge_index, Wq0, bq0, Wk0, bk0, W0, b0, Wq1, bq1, Wk1, bk1,
           W1, b1):
    f32 = jnp.float32
    xp = jnp.pad(x, ((0, NP - N), (0, 0)))
    row = edge_index[0]
    col = edge_index[1]
    loops = jnp.arange(N, dtype=jnp.int32)
    padi = 10000 + (jnp.arange(EP - E - N, dtype=jnp.int32) % 64)
    rowp = jnp.concatenate([row, loops, padi])
    colp = jnp.concatenate([col, loops, padi])
    idx_h = jnp.stack(
        [rowp.reshape(-1, W_WIN), colp.reshape(-1, W_WIN)], axis=1)

    wq16 = jnp.concatenate([Wq0, jnp.zeros((256, 8), f32)], 1)
    bq16 = jnp.concatenate([bq0, jnp.zeros((8,), f32)]).reshape(1, 16)
    wkv80 = jnp.concatenate([Wk0, jnp.zeros((256, 8), f32), W0], 1)
    bkv80 = jnp.concatenate([bk0, jnp.zeros((72,), f32)]).reshape(1, 80)

    wq1p = jnp.concatenate([Wq1, jnp.zeros((64, 15), f32)], 1)
    bq1p = jnp.concatenate([bq1, jnp.zeros((15,), f32)]).reshape(1, 16)
    wkv1p = jnp.concatenate(
        [Wk1, jnp.zeros((64, 7), f32), W1, jnp.zeros((64, 1), f32)], 1)
    bkv1p = jnp.concatenate(
        [bk1, jnp.zeros((6,), f32), jnp.ones((1,), f32),
         jnp.zeros((8,), f32)]).reshape(1, 16)
    b1p = jnp.concatenate([b1, jnp.zeros((1,), f32)]).reshape(1, 8)
    b0r = b0.reshape(1, 64)

    blk = 1024
    grid = NP // blk

    tq0, tkv0 = pl.pallas_call(
        _prep_kernel,
        grid=(grid,),
        in_specs=[
            pl.BlockSpec((blk, 256), lambda i: (i, 0)),
            pl.BlockSpec((256, 16), lambda i: (0, 0)),
            pl.BlockSpec((1, 16), lambda i: (0, 0)),
            pl.BlockSpec((256, 80), lambda i: (0, 0)),
            pl.BlockSpec((1, 80), lambda i: (0, 0)),
        ],
        out_specs=[
            pl.BlockSpec((blk, 16), lambda i: (i, 0)),
            pl.BlockSpec((blk, 80), lambda i: (i, 0)),
        ],
        out_shape=[
            jax.ShapeDtypeStruct((NP, 16), f32),
            jax.ShapeDtypeStruct((NP, 80), f32),
        ],
    )(xp, wq16, bq16, wkv80, bkv80)

    acc0 = _edge_sc_80(idx_h, tq0, tkv0)

    tq1, tkv1 = pl.pallas_call(
        _mid_kernel,
        grid=(grid,),
        in_specs=[
            pl.BlockSpec((2, blk, 80), lambda i: (0, i, 0)),
            pl.BlockSpec((1, 64), lambda i: (0, 0)),
            pl.BlockSpec((64, 16), lambda i: (0, 0)),
            pl.BlockSpec((1, 16), lambda i: (0, 0)),
            pl.BlockSpec((64, 16), lambda i: (0, 0)),
            pl.BlockSpec((1, 16), lambda i: (0, 0)),
        ],
        out_specs=[
            pl.BlockSpec((blk, 16), lambda i: (i, 0)),
            pl.BlockSpec((blk, 16), lambda i: (i, 0)),
        ],
        out_shape=[
            jax.ShapeDtypeStruct((NP, 16), f32),
            jax.ShapeDtypeStruct((NP, 16), f32),
        ],
    )(acc0, b0r, wq1p, bq1p, wkv1p, bkv1p)

    acc1 = _edge_sc_16(idx_h, tq1, tkv1)

    outp = pl.pallas_call(
        _final_kernel,
        grid=(grid,),
        in_specs=[
            pl.BlockSpec((2, blk, 16), lambda i: (0, i, 0)),
            pl.BlockSpec((1, 8), lambda i: (0, 0)),
        ],
        out_specs=pl.BlockSpec((blk, 8), lambda i: (i, 0)),
        out_shape=jax.ShapeDtypeStruct((NP, 8), f32),
    )(acc1, b1p)

    return outp[:N, :7]


# speedup vs baseline: 111.8813x; 1.1183x over previous
import functools

import jax
import jax.numpy as jnp
from jax import lax
from jax.experimental import pallas as pl
from jax.experimental.pallas import tpu as pltpu
from jax.experimental.pallas import tpu_sc as plsc

N = 10000
E = 160000
NP = 10240
W_WIN = 192
NC = 2
NS = 16
N_WIN = 28
EP = NC * NS * N_WIN * W_WIN
ROWS_PER_TILE = NP // NS
_CHUNK = 128


def _take16(x, idx):
    return lax.gather(
        x, idx[:, None],
        dimension_numbers=lax.GatherDimensionNumbers(
            offset_dims=(), collapsed_slice_dims=(0,), start_index_map=(0,)),
        slice_sizes=(1,),
        mode=lax.GatherScatterMode.PROMISE_IN_BOUNDS)


def _edge_sc_kernel(width):
    mesh = plsc.VectorSubcoreMesh(core_axis_name="c", subcore_axis_name="s")

    @functools.partial(
        pl.kernel,
        mesh=mesh,
        compiler_params=pltpu.CompilerParams(use_tc_tiling_on_sc=False),
        out_type=jax.ShapeDtypeStruct((NC, NP, width), jnp.float32),
        scratch_types=[
            pltpu.VMEM((2, 2, W_WIN), jnp.int32),
            pltpu.VMEM((2, 2, W_WIN // 2), jnp.int32),
            pltpu.VMEM((2, W_WIN, 16), jnp.float32),
            pltpu.VMEM((2, W_WIN, width), jnp.float32),
            pltpu.VMEM((2, W_WIN, width), jnp.float32),
            pltpu.VMEM_SHARED((NP, width), jnp.float32),
            pltpu.SemaphoreType.DMA((2,)),
            pltpu.SemaphoreType.DMA((2,)),
            pltpu.SemaphoreType.DMA((2,)),
            pltpu.SemaphoreType.DMA((2,)),
        ],
    )
    def k(idxr_h, idxc_h, tq_h, tkv_h, out_h, idxb, sidx, qr, kvr, upd, acc,
          sem_i, sem_q, sem_kv, sem_s):
        cid = lax.axis_index("c")
        sid = lax.axis_index("s")
        wid = cid * NS + sid
        r0 = sid * ROWS_PER_TILE
        H = W_WIN // 2
        base_e = wid * (N_WIN * W_WIN)

        zero16 = jnp.zeros((16,), jnp.float32)

        def zr(i, c):
            for j in range(width // 16):
                upd[0, i, pl.ds(16 * j, 16)] = zero16
            return c

        lax.fori_loop(0, _CHUNK, zr, 0)
        for j in range(ROWS_PER_TILE // _CHUNK):
            pltpu.sync_copy(upd.at[0, pl.ds(0, _CHUNK)],
                            acc.at[pl.ds(r0 + j * _CHUNK, _CHUNK)])
        plsc.subcore_barrier()

        lane = lax.iota(jnp.int32, 16)
        zidx = jnp.zeros((16,), jnp.int32)

        def start_idx(w, p):
            off = base_e + w * W_WIN
            pltpu.async_copy(idxr_h.at[pl.ds(off, W_WIN)], idxb.at[p, 0],
                             sem_i.at[p])
            pltpu.async_copy(idxc_h.at[pl.ds(off, W_WIN)], idxb.at[p, 1],
                             sem_i.at[p])

        def wait_idx(p):
            pltpu.make_async_copy(idxr_h.at[pl.ds(0, W_WIN)], idxb.at[p, 0],
                                  sem_i.at[p]).wait()
            pltpu.make_async_copy(idxc_h.at[pl.ds(0, W_WIN)], idxb.at[p, 1],
                                  sem_i.at[p]).wait()

        def start_gathers(w, p):
            for s in range(2):
                pltpu.async_copy(tq_h.at[idxb.at[p, 0, pl.ds(s * H, H)]],
                                 qr.at[p, pl.ds(s * H, H)], sem_q.at[p])
                pltpu.async_copy(tkv_h.at[idxb.at[p, 1, pl.ds(s * H, H)]],
                                 kvr.at[p, pl.ds(s * H, H)], sem_kv.at[p])

        def wait_gathers(p):
            for s in range(2):
                pltpu.make_async_copy(tq_h.at[idxb.at[p, 0, pl.ds(s * H, H)]],
                                      qr.at[p, pl.ds(s * H, H)],
                                      sem_q.at[p]).wait()
                pltpu.make_async_copy(tkv_h.at[idxb.at[p, 1, pl.ds(s * H, H)]],
                                      kvr.at[p, pl.ds(s * H, H)],
                                      sem_kv.at[p]).wait()

        def start_scatter(p):
            for s in range(2):
                pltpu.async_copy(upd.at[p, pl.ds(s * H, H)],
                                 acc.at[sidx.at[p, s]], sem_s.at[p],
                                 add=True)

        def wait_scatter(p):
            for s in range(2):
                pltpu.make_async_copy(upd.at[p, pl.ds(s * H, H)],
                                      acc.at[sidx.at[p, s]],
                                      sem_s.at[p]).wait()

        def compute(p):
            def edge(e, c2):
                for u in range(4):
                    eu = 4 * e + u
                    q = qr[p, eu]
                    if width == 80:
                        kv0 = kvr[p, eu, pl.ds(0, 16)]
                        ex = jnp.exp(q * kv0)
                        upd[p, eu, pl.ds(0, 16)] = ex
                        for j in range(4):
                            bj = _take16(ex, (lane >> 3) + 2 * j)
                            vj = kvr[p, eu, pl.ds(16 + 16 * j, 16)]
                            upd[p, eu, pl.ds(16 + 16 * j, 16)] = bj * vj
                    else:
                        kv = kvr[p, eu]
                        ex = jnp.exp(q * kv)
                        b0v = _take16(ex, zidx)
                        upd[p, eu] = b0v * kv
                return c2

            lax.fori_loop(0, W_WIN // 4, edge, 0)

        def body(w, p):
            @pl.when(w >= 2)
            def _():
                wait_scatter(p)

            wait_gathers(p)
            for s in range(2):
                for i in range(H // 16):
                    sidx[p, s, pl.ds(16 * i, 16)] = (
                        idxb[p, 0, pl.ds(s * H + 16 * i, 16)])

            @pl.when(w + 1 < N_WIN)
            def _():
                wait_idx(1 - p)
                start_gathers(w + 1, 1 - p)

            @pl.when(w + 2 < N_WIN)
            def _():
                start_idx(w + 2, p)

            compute(p)
            start_scatter(p)

        start_idx(0, 0)
        start_idx(1, 1)
        wait_idx(0)
        start_gathers(0, 0)

        def outer(g, carry):
            body(2 * g, 0)
            body(2 * g + 1, 1)
            return carry

        lax.fori_loop(0, N_WIN // 2, outer, 0)
        if N_WIN % 2:
            body(N_WIN - 1, 0)
        wait_scatter(0)
        wait_scatter(1)
        plsc.subcore_barrier()

        for j in range(ROWS_PER_TILE // _CHUNK):
            sl = pl.ds(r0 + j * _CHUNK, _CHUNK)
            pltpu.sync_copy(acc.at[sl], upd.at[0, pl.ds(0, _CHUNK)])
            pltpu.sync_copy(upd.at[0, pl.ds(0, _CHUNK)], out_h.at[cid, sl])

    return k


_edge_sc_80 = _edge_sc_kernel(80)
_edge_sc_16 = _edge_sc_kernel(16)

_HIGH = None
_HIGHEST = lax.Precision.HIGHEST
def _prep_kernel(x_ref, wq_ref, bq_ref, wkv_ref, bkv_ref, tq_ref, tkv_ref):
    x = x_ref[...]
    tq_ref[...] = jax.nn.relu(
        jnp.dot(x, wq_ref[...], preferred_element_type=jnp.float32,
                precision=None) + bq_ref[...])
    t = jnp.dot(x, wkv_ref[...], preferred_element_type=jnp.float32,
                precision=None) + bkv_ref[...]
    ci = lax.broadcasted_iota(jnp.int32, t.shape, 1)
    tkv_ref[...] = jnp.where(ci < 16, jax.nn.relu(t), t)


def _mid_kernel(a_ref, b0_ref, wq1_ref, bq1_ref, wkv1_ref, bkv1_ref,
                tq1_ref, tkv1_ref):
    s = a_ref[0] + a_ref[1]
    d8 = s[:, 0:8]
    r = lax.broadcasted_iota(jnp.int32, (8, 64), 0)
    c = lax.broadcasted_iota(jnp.int32, (8, 64), 1)
    onehot = (r == c // 8).astype(jnp.float32)
    d64 = jnp.dot(d8, onehot, preferred_element_type=jnp.float32,
                  precision=_HIGHEST)
    h = jax.nn.relu(s[:, 16:80] / (d64 + 1e-16) + b0_ref[...])
    tq1_ref[...] = jax.nn.relu(
        jnp.dot(h, wq1_ref[...], preferred_element_type=jnp.float32,
                precision=_HIGHEST) + bq1_ref[...])
    t = jnp.dot(h, wkv1_ref[...], preferred_element_type=jnp.float32,
                precision=_HIGHEST) + bkv1_ref[...]
    ci = lax.broadcasted_iota(jnp.int32, t.shape, 1)
    tkv1_ref[...] = jnp.where(ci < 8, jax.nn.relu(t), t)


def _final_kernel(a_ref, b1_ref, out_ref):
    s = a_ref[0] + a_ref[1]
    denom = s[:, 7:8]
    msg = s[:, 8:15]
    out_ref[...] = msg / (denom + 1e-16) + b1_ref[...]


def kernel(x, edge_index, Wq0, bq0, Wk0, bk0, W0, b0, Wq1, bq1, Wk1, bk1,
           W1, b1):
    f32 = jnp.float32
    wq16 = jnp.concatenate([Wq0, jnp.zeros((256, 8), f32)], 1)
    bq16 = jnp.concatenate([bq0, jnp.zeros((8,), f32)]).reshape(1, 16)
    wkv80 = jnp.concatenate([Wk0, jnp.zeros((256, 8), f32), W0], 1)
    bkv80 = jnp.concatenate([bk0, jnp.zeros((72,), f32)]).reshape(1, 80)

    wq1p = jnp.concatenate([Wq1, jnp.zeros((64, 15), f32)], 1)
    bq1p = jnp.concatenate([bq1, jnp.zeros((15,), f32)]).reshape(1, 16)
    wkv1p = jnp.concatenate(
        [Wk1, jnp.zeros((64, 7), f32), W1, jnp.zeros((64, 1), f32)], 1)
    bkv1p = jnp.concatenate(
        [bk1, jnp.zeros((6,), f32), jnp.ones((1,), f32),
         jnp.zeros((8,), f32)]).reshape(1, 16)
    b1p = b1.reshape(1, 7)
    b0r = b0.reshape(1, 64)

    te = jnp.arange(E, EP, dtype=jnp.int32)
    tailv = jnp.where(te < E + N, te - E, 10000 + lax.rem(te, 64))
    idxr_f = jnp.concatenate([edge_index[0], tailv])
    idxc_f = jnp.concatenate([edge_index[1], tailv])

    blk = 1000
    grid = N // blk

    tq0, tkv0 = pl.pallas_call(
        _prep_kernel,
        grid=(grid,),
        in_specs=[
            pl.BlockSpec((blk, 256), lambda i: (i, 0)),
            pl.BlockSpec((256, 16), lambda i: (0, 0)),
            pl.BlockSpec((1, 16), lambda i: (0, 0)),
            pl.BlockSpec((256, 80), lambda i: (0, 0)),
            pl.BlockSpec((1, 80), lambda i: (0, 0)),
        ],
        out_specs=[
            pl.BlockSpec((blk, 16), lambda i: (i, 0)),
            pl.BlockSpec((blk, 80), lambda i: (i, 0)),
        ],
        out_shape=[
            jax.ShapeDtypeStruct((NP, 16), f32),
            jax.ShapeDtypeStruct((NP, 80), f32),
        ],
    )(x, wq16, bq16, wkv80, bkv80)

    acc0 = _edge_sc_80(idxr_f, idxc_f, tq0, tkv0)

    blk2 = 1024
    grid2 = NP // blk2
    tq1, tkv1 = pl.pallas_call(
        _mid_kernel,
        grid=(grid2,),
        in_specs=[
            pl.BlockSpec((2, blk2, 80), lambda i: (0, i, 0)),
            pl.BlockSpec((1, 64), lambda i: (0, 0)),
            pl.BlockSpec((64, 16), lambda i: (0, 0)),
            pl.BlockSpec((1, 16), lambda i: (0, 0)),
            pl.BlockSpec((64, 16), lambda i: (0, 0)),
            pl.BlockSpec((1, 16), lambda i: (0, 0)),
        ],
        out_specs=[
            pl.BlockSpec((blk2, 16), lambda i: (i, 0)),
            pl.BlockSpec((blk2, 16), lambda i: (i, 0)),
        ],
        out_shape=[
            jax.ShapeDtypeStruct((NP, 16), f32),
            jax.ShapeDtypeStruct((NP, 16), f32),
        ],
    )(acc0, b0r, wq1p, bq1p, wkv1p, bkv1p)

    acc1 = _edge_sc_16(idxr_f, idxc_f, tq1, tkv1)

    outp = pl.pallas_call(
        _final_kernel,
        grid=(grid,),
        in_specs=[
            pl.BlockSpec((2, blk, 16), lambda i: (0, i, 0)),
            pl.BlockSpec((1, 7), lambda i: (0, 0)),
        ],
        out_specs=pl.BlockSpec((blk, 7), lambda i: (i, 0)),
        out_shape=jax.ShapeDtypeStruct((N, 7), f32),
    )(acc1, b1p)

    return outp


# speedup vs baseline: 113.2845x; 1.0125x over previous
import functools

import jax
import jax.numpy as jnp
from jax import lax
from jax.experimental import pallas as pl
from jax.experimental.pallas import tpu as pltpu
from jax.experimental.pallas import tpu_sc as plsc

N = 10000
E = 160000
NP = 10240
W_WIN = 160
NC = 2
NS = 16
N_WIN = 34
EP = NC * NS * N_WIN * W_WIN
EDGE_WINDOWS = E // W_WIN
ROWS_PER_TILE = NP // NS
_CHUNK = 128


def _take16(x, idx):
    return lax.gather(
        x, idx[:, None],
        dimension_numbers=lax.GatherDimensionNumbers(
            offset_dims=(), collapsed_slice_dims=(0,), start_index_map=(0,)),
        slice_sizes=(1,),
        mode=lax.GatherScatterMode.PROMISE_IN_BOUNDS)


def _edge_sc_kernel(width):
    mesh = plsc.VectorSubcoreMesh(core_axis_name="c", subcore_axis_name="s")

    @functools.partial(
        pl.kernel,
        mesh=mesh,
        compiler_params=pltpu.CompilerParams(use_tc_tiling_on_sc=False),
        out_type=jax.ShapeDtypeStruct((NC, NP, width), jnp.float32),
        scratch_types=[
            pltpu.VMEM((2, 2, W_WIN), jnp.int32),
            pltpu.VMEM((2, 2, W_WIN // 2), jnp.int32),
            pltpu.VMEM((2, W_WIN, 16), jnp.float32),
            pltpu.VMEM((2, W_WIN, width), jnp.float32),
            pltpu.VMEM((2, W_WIN, width), jnp.float32),
            pltpu.VMEM_SHARED((NP, width), jnp.float32),
            pltpu.SemaphoreType.DMA((2,)),
            pltpu.SemaphoreType.DMA((2,)),
            pltpu.SemaphoreType.DMA((2,)),
            pltpu.SemaphoreType.DMA((2,)),
        ],
    )
    def k(ei_h, tq_h, tkv_h, out_h, idxb, sidx, qr, kvr, upd, acc,
          sem_i, sem_q, sem_kv, sem_s):
        cid = lax.axis_index("c")
        sid = lax.axis_index("s")
        wid = cid * NS + sid
        r0 = sid * ROWS_PER_TILE
        H = W_WIN // 2

        zero16 = jnp.zeros((16,), jnp.float32)

        def zr(i, c):
            for j in range(width // 16):
                upd[0, i, pl.ds(16 * j, 16)] = zero16
            return c

        lax.fori_loop(0, _CHUNK, zr, 0)
        for j in range(ROWS_PER_TILE // _CHUNK):
            pltpu.sync_copy(upd.at[0, pl.ds(0, _CHUNK)],
                            acc.at[pl.ds(r0 + j * _CHUNK, _CHUNK)])
        plsc.subcore_barrier()

        lane = lax.iota(jnp.int32, 16)
        zidx = jnp.zeros((16,), jnp.int32)

        def start_idx(w, p):
            g = wid * N_WIN + w

            @pl.when(g < EDGE_WINDOWS)
            def _():
                off = g * W_WIN
                pltpu.async_copy(ei_h.at[0, pl.ds(off, W_WIN)],
                                 idxb.at[p, 0], sem_i.at[p])
                pltpu.async_copy(ei_h.at[1, pl.ds(off, W_WIN)],
                                 idxb.at[p, 1], sem_i.at[p])

        def ready_idx(w, p):
            g = wid * N_WIN + w

            @pl.when(g < EDGE_WINDOWS)
            def _():
                for r in range(2):
                    pltpu.make_async_copy(ei_h.at[r, pl.ds(0, W_WIN)],
                                          idxb.at[p, r], sem_i.at[p]).wait()

            @pl.when(g >= EDGE_WINDOWS)
            def _():
                for kk in range(W_WIN // 16):
                    ev = g * W_WIN + 16 * kk + lane
                    v = jnp.where(ev < E + N, ev - E, 10000 + (ev & 63))
                    idxb[p, 0, pl.ds(16 * kk, 16)] = v
                    idxb[p, 1, pl.ds(16 * kk, 16)] = v

        def start_gathers(w, p):
            for s in range(2):
                pltpu.async_copy(tq_h.at[idxb.at[p, 0, pl.ds(s * H, H)]],
                                 qr.at[p, pl.ds(s * H, H)], sem_q.at[p])
                pltpu.async_copy(tkv_h.at[idxb.at[p, 1, pl.ds(s * H, H)]],
                                 kvr.at[p, pl.ds(s * H, H)], sem_kv.at[p])

        def wait_gathers(p):
            for s in range(2):
                pltpu.make_async_copy(tq_h.at[idxb.at[p, 0, pl.ds(s * H, H)]],
                                      qr.at[p, pl.ds(s * H, H)],
                                      sem_q.at[p]).wait()
                pltpu.make_async_copy(tkv_h.at[idxb.at[p, 1, pl.ds(s * H, H)]],
                                      kvr.at[p, pl.ds(s * H, H)],
                                      sem_kv.at[p]).wait()

        def start_scatter(p):
            for s in range(2):
                pltpu.async_copy(upd.at[p, pl.ds(s * H, H)],
                                 acc.at[sidx.at[p, s]], sem_s.at[p],
                                 add=True)

        def wait_scatter(p):
            for s in range(2):
                pltpu.make_async_copy(upd.at[p, pl.ds(s * H, H)],
                                      acc.at[sidx.at[p, s]],
                                      sem_s.at[p]).wait()

        def compute(p):
            def edge(e, c2):
                for u in range(4):
                    eu = 4 * e + u
                    q = qr[p, eu]
                    if width == 80:
                        kv0 = kvr[p, eu, pl.ds(0, 16)]
                        ex = jnp.exp(q * kv0)
                        upd[p, eu, pl.ds(0, 16)] = ex
                        for j in range(4):
                            bj = _take16(ex, (lane >> 3) + 2 * j)
                            vj = kvr[p, eu, pl.ds(16 + 16 * j, 16)]
                            upd[p, eu, pl.ds(16 + 16 * j, 16)] = bj * vj
                    else:
                        kv = kvr[p, eu]
                        ex = jnp.exp(q * kv)
                        b0v = _take16(ex, zidx)
                        upd[p, eu] = b0v * kv
                return c2

            lax.fori_loop(0, W_WIN // 4, edge, 0)

        def body(w, p):
            @pl.when(w >= 2)
            def _():
                wait_scatter(p)

            wait_gathers(p)
            for s in range(2):
                for i in range(H // 16):
                    sidx[p, s, pl.ds(16 * i, 16)] = (
                        idxb[p, 0, pl.ds(s * H + 16 * i, 16)])

            @pl.when(w + 1 < N_WIN)
            def _():
                ready_idx(w + 1, 1 - p)
                start_gathers(w + 1, 1 - p)

            @pl.when(w + 2 < N_WIN)
            def _():
                start_idx(w + 2, p)

            compute(p)
            start_scatter(p)

        start_idx(0, 0)
        start_idx(1, 1)
        ready_idx(0, 0)
        start_gathers(0, 0)

        def outer(g, carry):
            body(2 * g, 0)
            body(2 * g + 1, 1)
            return carry

        lax.fori_loop(0, N_WIN // 2, outer, 0)
        if N_WIN % 2:
            body(N_WIN - 1, 0)
        wait_scatter(0)
        wait_scatter(1)
        plsc.subcore_barrier()

        for j in range(ROWS_PER_TILE // _CHUNK):
            sl = pl.ds(r0 + j * _CHUNK, _CHUNK)
            pltpu.sync_copy(acc.at[sl], upd.at[0, pl.ds(0, _CHUNK)])
            pltpu.sync_copy(upd.at[0, pl.ds(0, _CHUNK)], out_h.at[cid, sl])

    return k


_edge_sc_80 = _edge_sc_kernel(80)
_edge_sc_16 = _edge_sc_kernel(16)

_HIGH = None
_HIGHEST = lax.Precision.HIGHEST
def _prep_kernel(x_ref, wq_ref, bq_ref, wkv_ref, bkv_ref, tq_ref, tkv_ref):
    x = x_ref[...]
    tq_ref[...] = jax.nn.relu(
        jnp.dot(x, wq_ref[...], preferred_element_type=jnp.float32,
                precision=None) + bq_ref[...])
    t = jnp.dot(x, wkv_ref[...], preferred_element_type=jnp.float32,
                precision=None) + bkv_ref[...]
    ci = lax.broadcasted_iota(jnp.int32, t.shape, 1)
    tkv_ref[...] = jnp.where(ci < 16, jax.nn.relu(t), t)


def _mid_kernel(a_ref, b0_ref, wq1_ref, bq1_ref, wkv1_ref, bkv1_ref,
                tq1_ref, tkv1_ref):
    s = a_ref[0] + a_ref[1]
    d8 = s[:, 0:8]
    r = lax.broadcasted_iota(jnp.int32, (8, 64), 0)
    c = lax.broadcasted_iota(jnp.int32, (8, 64), 1)
    onehot = (r == c // 8).astype(jnp.float32)
    d64 = jnp.dot(d8, onehot, preferred_element_type=jnp.float32,
                  precision=_HIGHEST)
    h = jax.nn.relu(s[:, 16:80] / (d64 + 1e-16) + b0_ref[...])
    tq1_ref[...] = jax.nn.relu(
        jnp.dot(h, wq1_ref[...], preferred_element_type=jnp.float32,
                precision=_HIGHEST) + bq1_ref[...])
    t = jnp.dot(h, wkv1_ref[...], preferred_element_type=jnp.float32,
                precision=_HIGHEST) + bkv1_ref[...]
    ci = lax.broadcasted_iota(jnp.int32, t.shape, 1)
    tkv1_ref[...] = jnp.where(ci < 8, jax.nn.relu(t), t)


def _final_kernel(a_ref, b1_ref, out_ref):
    s = a_ref[0] + a_ref[1]
    denom = s[:, 7:8]
    msg = s[:, 8:15]
    out_ref[...] = msg / (denom + 1e-16) + b1_ref[...]


def kernel(x, edge_index, Wq0, bq0, Wk0, bk0, W0, b0, Wq1, bq1, Wk1, bk1,
           W1, b1):
    f32 = jnp.float32
    wq16 = jnp.concatenate([Wq0, jnp.zeros((256, 8), f32)], 1)
    bq16 = jnp.concatenate([bq0, jnp.zeros((8,), f32)]).reshape(1, 16)
    wkv80 = jnp.concatenate([Wk0, jnp.zeros((256, 8), f32), W0], 1)
    bkv80 = jnp.concatenate([bk0, jnp.zeros((72,), f32)]).reshape(1, 80)

    wq1p = jnp.concatenate([Wq1, jnp.zeros((64, 15), f32)], 1)
    bq1p = jnp.concatenate([bq1, jnp.zeros((15,), f32)]).reshape(1, 16)
    wkv1p = jnp.concatenate(
        [Wk1, jnp.zeros((64, 7), f32), W1, jnp.zeros((64, 1), f32)], 1)
    bkv1p = jnp.concatenate(
        [bk1, jnp.zeros((6,), f32), jnp.ones((1,), f32),
         jnp.zeros((8,), f32)]).reshape(1, 16)
    b1p = b1.reshape(1, 7)
    b0r = b0.reshape(1, 64)


    blk = 1000
    grid = N // blk

    tq0, tkv0 = pl.pallas_call(
        _prep_kernel,
        grid=(grid,),
        in_specs=[
            pl.BlockSpec((blk, 256), lambda i: (i, 0)),
            pl.BlockSpec((256, 16), lambda i: (0, 0)),
            pl.BlockSpec((1, 16), lambda i: (0, 0)),
            pl.BlockSpec((256, 80), lambda i: (0, 0)),
            pl.BlockSpec((1, 80), lambda i: (0, 0)),
        ],
        out_specs=[
            pl.BlockSpec((blk, 16), lambda i: (i, 0)),
            pl.BlockSpec((blk, 80), lambda i: (i, 0)),
        ],
        out_shape=[
            jax.ShapeDtypeStruct((NP, 16), f32),
            jax.ShapeDtypeStruct((NP, 80), f32),
        ],
    )(x, wq16, bq16, wkv80, bkv80)

    acc0 = _edge_sc_80(edge_index, tq0, tkv0)

    blk2 = 1024
    grid2 = NP // blk2
    tq1, tkv1 = pl.pallas_call(
        _mid_kernel,
        grid=(grid2,),
        in_specs=[
            pl.BlockSpec((2, blk2, 80), lambda i: (0, i, 0)),
            pl.BlockSpec((1, 64), lambda i: (0, 0)),
            pl.BlockSpec((64, 16), lambda i: (0, 0)),
            pl.BlockSpec((1, 16), lambda i: (0, 0)),
            pl.BlockSpec((64, 16), lambda i: (0, 0)),
            pl.BlockSpec((1, 16), lambda i: (0, 0)),
        ],
        out_specs=[
            pl.BlockSpec((blk2, 16), lambda i: (i, 0)),
            pl.BlockSpec((blk2, 16), lambda i: (i, 0)),
        ],
        out_shape=[
            jax.ShapeDtypeStruct((NP, 16), f32),
            jax.ShapeDtypeStruct((NP, 16), f32),
        ],
    )(acc0, b0r, wq1p, bq1p, wkv1p, bkv1p)

    acc1 = _edge_sc_16(edge_index, tq1, tkv1)

    outp = pl.pallas_call(
        _final_kernel,
        grid=(grid,),
        in_specs=[
            pl.BlockSpec((2, blk, 16), lambda i: (0, i, 0)),
            pl.BlockSpec((1, 7), lambda i: (0, 0)),
        ],
        out_specs=pl.BlockSpec((blk, 7), lambda i: (i, 0)),
        out_shape=jax.ShapeDtypeStruct((N, 7), f32),
    )(acc1, b1p)

    return outp


# speedup vs baseline: 117.2869x; 1.0353x over previous
import functools

import jax
import jax.numpy as jnp
from jax import lax
from jax.experimental import pallas as pl
from jax.experimental.pallas import tpu as pltpu
from jax.experimental.pallas import tpu_sc as plsc

N = 10000
E = 160000
NP = 10240
W_WIN = 160
NC = 2
NS = 16
N_WIN = 34
EP = NC * NS * N_WIN * W_WIN
EDGE_WINDOWS = E // W_WIN
ROWS_PER_TILE = NP // NS
_CHUNK = 128


def _take16(x, idx):
    return lax.gather(
        x, idx[:, None],
        dimension_numbers=lax.GatherDimensionNumbers(
            offset_dims=(), collapsed_slice_dims=(0,), start_index_map=(0,)),
        slice_sizes=(1,),
        mode=lax.GatherScatterMode.PROMISE_IN_BOUNDS)


def _edge_sc_kernel(width):
    mesh = plsc.VectorSubcoreMesh(core_axis_name="c", subcore_axis_name="s")

    @functools.partial(
        pl.kernel,
        mesh=mesh,
        compiler_params=pltpu.CompilerParams(use_tc_tiling_on_sc=False),
        out_type=jax.ShapeDtypeStruct((NC, NP, width), jnp.float32),
        scratch_types=[
            pltpu.VMEM((2, 2, W_WIN), jnp.int32),
            pltpu.VMEM((2, 2, W_WIN // 2), jnp.int32),
            pltpu.VMEM((2, W_WIN, 16), jnp.float32),
            pltpu.VMEM((2, W_WIN, width), jnp.float32),
            pltpu.VMEM((2, W_WIN, width), jnp.float32),
            pltpu.VMEM_SHARED((NP, width), jnp.float32),
            pltpu.SemaphoreType.DMA((2,)),
            pltpu.SemaphoreType.DMA((2,)),
            pltpu.SemaphoreType.DMA((2,)),
            pltpu.SemaphoreType.DMA((2,)),
        ],
    )
    def k(ei_h, tq_h, tkv_h, out_h, idxb, sidx, qr, kvr, upd, acc,
          sem_i, sem_q, sem_kv, sem_s):
        cid = lax.axis_index("c")
        sid = lax.axis_index("s")
        wid = cid * NS + sid
        r0 = sid * ROWS_PER_TILE
        H = W_WIN // 2

        zero16 = jnp.zeros((16,), jnp.float32)

        def zr(i, c):
            for j in range(width // 16):
                upd[0, i, pl.ds(16 * j, 16)] = zero16
            return c

        lax.fori_loop(0, _CHUNK, zr, 0)
        for j in range(ROWS_PER_TILE // _CHUNK):
            pltpu.sync_copy(upd.at[0, pl.ds(0, _CHUNK)],
                            acc.at[pl.ds(r0 + j * _CHUNK, _CHUNK)])
        plsc.subcore_barrier()

        lane = lax.iota(jnp.int32, 16)
        eight = jnp.full((16,), 8, jnp.int32)

        def start_idx(w, p):
            g = wid * N_WIN + w

            @pl.when(g < EDGE_WINDOWS)
            def _():
                off = g * W_WIN
                pltpu.async_copy(ei_h.at[0, pl.ds(off, W_WIN)],
                                 idxb.at[p, 0], sem_i.at[p])
                pltpu.async_copy(ei_h.at[1, pl.ds(off, W_WIN)],
                                 idxb.at[p, 1], sem_i.at[p])

        def ready_idx(w, p):
            g = wid * N_WIN + w

            @pl.when(g < EDGE_WINDOWS)
            def _():
                for r in range(2):
                    pltpu.make_async_copy(ei_h.at[r, pl.ds(0, W_WIN)],
                                          idxb.at[p, r], sem_i.at[p]).wait()

            @pl.when(g >= EDGE_WINDOWS)
            def _():
                for kk in range(W_WIN // 16):
                    ev = g * W_WIN + 16 * kk + lane
                    v = jnp.where(ev < E + N, ev - E, 10000 + (ev & 63))
                    idxb[p, 0, pl.ds(16 * kk, 16)] = v
                    idxb[p, 1, pl.ds(16 * kk, 16)] = v

        def start_gathers(w, p):
            for s in range(2):
                pltpu.async_copy(tq_h.at[idxb.at[p, 0, pl.ds(s * H, H)]],
                                 qr.at[p, pl.ds(s * H, H)], sem_q.at[p])
                pltpu.async_copy(tkv_h.at[idxb.at[p, 1, pl.ds(s * H, H)]],
                                 kvr.at[p, pl.ds(s * H, H)], sem_kv.at[p])

        def wait_gathers(p):
            for s in range(2):
                pltpu.make_async_copy(tq_h.at[idxb.at[p, 0, pl.ds(s * H, H)]],
                                      qr.at[p, pl.ds(s * H, H)],
                                      sem_q.at[p]).wait()
                pltpu.make_async_copy(tkv_h.at[idxb.at[p, 1, pl.ds(s * H, H)]],
                                      kvr.at[p, pl.ds(s * H, H)],
                                      sem_kv.at[p]).wait()

        def start_scatter(p):
            for s in range(2):
                pltpu.async_copy(upd.at[p, pl.ds(s * H, H)],
                                 acc.at[sidx.at[p, s]], sem_s.at[p],
                                 add=True)

        def wait_scatter(p):
            for s in range(2):
                pltpu.make_async_copy(upd.at[p, pl.ds(s * H, H)],
                                      acc.at[sidx.at[p, s]],
                                      sem_s.at[p]).wait()

        def compute(p):
            def edge(e, c2):
                for u in range(4):
                    eu = 4 * e + u
                    q = qr[p, eu]
                    if width == 80:
                        kvk = kvr[p, eu, pl.ds(64, 16)]
                        ex = jnp.exp(q * kvk)
                        upd[p, eu, pl.ds(64, 16)] = ex
                        for j in range(4):
                            bj = _take16(ex, (lane >> 3) + 2 * j)
                            vj = kvr[p, eu, pl.ds(16 * j, 16)]
                            upd[p, eu, pl.ds(16 * j, 16)] = bj * vj
                    else:
                        kv = kvr[p, eu]
                        ex = jnp.exp(q * kv)
                        b0v = _take16(ex, eight)
                        upd[p, eu] = b0v * kv
                return c2

            lax.fori_loop(0, W_WIN // 4, edge, 0)

        def body(w, p):
            @pl.when(w >= 2)
            def _():
                wait_scatter(p)

            wait_gathers(p)
            for s in range(2):
                for i in range(H // 16):
                    sidx[p, s, pl.ds(16 * i, 16)] = (
                        idxb[p, 0, pl.ds(s * H + 16 * i, 16)])

            @pl.when(w + 1 < N_WIN)
            def _():
                ready_idx(w + 1, 1 - p)
                start_gathers(w + 1, 1 - p)

            @pl.when(w + 2 < N_WIN)
            def _():
                start_idx(w + 2, p)

            compute(p)
            start_scatter(p)

        start_idx(0, 0)
        start_idx(1, 1)
        ready_idx(0, 0)
        start_gathers(0, 0)

        def outer(g, carry):
            body(2 * g, 0)
            body(2 * g + 1, 1)
            return carry

        lax.fori_loop(0, N_WIN // 2, outer, 0)
        if N_WIN % 2:
            body(N_WIN - 1, 0)
        wait_scatter(0)
        wait_scatter(1)
        plsc.subcore_barrier()

        for j in range(ROWS_PER_TILE // _CHUNK):
            sl = pl.ds(r0 + j * _CHUNK, _CHUNK)
            pltpu.sync_copy(acc.at[sl], upd.at[0, pl.ds(0, _CHUNK)])
            pltpu.sync_copy(upd.at[0, pl.ds(0, _CHUNK)], out_h.at[cid, sl])

    return k


_edge_sc_80 = _edge_sc_kernel(80)
_edge_sc_16 = _edge_sc_kernel(16)

_HIGH = None
_HIGHEST = lax.Precision.HIGHEST
def _prep_kernel(x_ref, wq_ref, bq_ref, wkv_ref, bkv_ref, tq_ref, tkv_ref):
    x = x_ref[...]
    tq_ref[...] = jax.nn.relu(
        jnp.dot(x, wq_ref[...], preferred_element_type=jnp.float32,
                precision=None) + bq_ref[...])
    t = jnp.dot(x, wkv_ref[...], preferred_element_type=jnp.float32,
                precision=None) + bkv_ref[...]
    ci = lax.broadcasted_iota(jnp.int32, t.shape, 1)
    tkv_ref[...] = jnp.where(ci >= 64, jax.nn.relu(t), t)


def _mid_kernel(a_ref, b0_ref, wq1_ref, bq1_ref, wkv1_ref, bkv1_ref,
                tq1_ref, tkv1_ref):
    s = a_ref[0] + a_ref[1]
    d8 = s[:, 64:72]
    r = lax.broadcasted_iota(jnp.int32, (8, 64), 0)
    c = lax.broadcasted_iota(jnp.int32, (8, 64), 1)
    onehot = (r == c // 8).astype(jnp.float32)
    d64 = jnp.dot(d8, onehot, preferred_element_type=jnp.float32,
                  precision=_HIGHEST)
    h = jax.nn.relu(s[:, 0:64] / (d64 + 1e-16) + b0_ref[...])
    tq1_ref[...] = jax.nn.relu(
        jnp.dot(h, wq1_ref[...], preferred_element_type=jnp.float32,
                precision=_HIGHEST) + bq1_ref[...])
    t = jnp.dot(h, wkv1_ref[...], preferred_element_type=jnp.float32,
                precision=_HIGHEST) + bkv1_ref[...]
    ci = lax.broadcasted_iota(jnp.int32, t.shape, 1)
    tkv1_ref[...] = jnp.where(ci >= 7, jax.nn.relu(t), t)


def _final_kernel(a_ref, b1_ref, out_ref):
    s = a_ref[0] + a_ref[1]
    r = lax.broadcasted_iota(jnp.int32, (128, 128), 0)
    c = lax.broadcasted_iota(jnp.int32, (128, 128), 1)
    md = (r == (c // 16) * 16 + 7).astype(jnp.float32)
    dvec = jnp.dot(s, md, preferred_element_type=jnp.float32,
                   precision=_HIGHEST)
    out_ref[...] = s / (dvec + 1e-16) + b1_ref[...]


def kernel(x, edge_index, Wq0, bq0, Wk0, bk0, W0, b0, Wq1, bq1, Wk1, bk1,
           W1, b1):
    f32 = jnp.float32
    wq16 = jnp.concatenate([Wq0, jnp.zeros((256, 8), f32)], 1)
    bq16 = jnp.concatenate([bq0, jnp.zeros((8,), f32)]).reshape(1, 16)
    wkv80 = jnp.concatenate([W0, Wk0, jnp.zeros((256, 8), f32)], 1)
    bkv80 = jnp.concatenate(
        [jnp.zeros((64,), f32), bk0, jnp.zeros((8,), f32)]).reshape(1, 80)

    wq1p = jnp.concatenate(
        [jnp.zeros((64, 8), f32), Wq1, jnp.zeros((64, 7), f32)], 1)
    bq1p = jnp.concatenate(
        [jnp.zeros((8,), f32), bq1, jnp.zeros((7,), f32)]).reshape(1, 16)
    wkv1p = jnp.concatenate(
        [W1, jnp.zeros((64, 1), f32), Wk1, jnp.zeros((64, 7), f32)], 1)
    bkv1p = jnp.concatenate(
        [jnp.zeros((7,), f32), jnp.ones((1,), f32), bk1,
         jnp.zeros((7,), f32)]).reshape(1, 16)
    b1p = jnp.tile(jnp.concatenate([b1, jnp.zeros((9,), f32)]),
                   8).reshape(1, 128)
    b0r = b0.reshape(1, 64)


    blk = 1000
    grid = N // blk

    tq0, tkv0 = pl.pallas_call(
        _prep_kernel,
        grid=(grid,),
        in_specs=[
            pl.BlockSpec((blk, 256), lambda i: (i, 0)),
            pl.BlockSpec((256, 16), lambda i: (0, 0)),
            pl.BlockSpec((1, 16), lambda i: (0, 0)),
            pl.BlockSpec((256, 80), lambda i: (0, 0)),
            pl.BlockSpec((1, 80), lambda i: (0, 0)),
        ],
        out_specs=[
            pl.BlockSpec((blk, 16), lambda i: (i, 0)),
            pl.BlockSpec((blk, 80), lambda i: (i, 0)),
        ],
        out_shape=[
            jax.ShapeDtypeStruct((NP, 16), f32),
            jax.ShapeDtypeStruct((NP, 80), f32),
        ],
    )(x, wq16, bq16, wkv80, bkv80)

    acc0 = _edge_sc_80(edge_index, tq0, tkv0)

    blk2 = 1024
    grid2 = NP // blk2
    tq1, tkv1 = pl.pallas_call(
        _mid_kernel,
        grid=(grid2,),
        in_specs=[
            pl.BlockSpec((2, blk2, 80), lambda i: (0, i, 0)),
            pl.BlockSpec((1, 64), lambda i: (0, 0)),
            pl.BlockSpec((64, 16), lambda i: (0, 0)),
            pl.BlockSpec((1, 16), lambda i: (0, 0)),
            pl.BlockSpec((64, 16), lambda i: (0, 0)),
            pl.BlockSpec((1, 16), lambda i: (0, 0)),
        ],
        out_specs=[
            pl.BlockSpec((blk2, 16), lambda i: (i, 0)),
            pl.BlockSpec((blk2, 16), lambda i: (i, 0)),
        ],
        out_shape=[
            jax.ShapeDtypeStruct((NP, 16), f32),
            jax.ShapeDtypeStruct((NP, 16), f32),
        ],
    )(acc0, b0r, wq1p, bq1p, wkv1p, bkv1p)

    acc1 = _edge_sc_16(edge_index, tq1, tkv1)

    npk = NP // 8
    acc1v = acc1.reshape(NC, npk, 128)
    blk3 = 128
    outp = pl.pallas_call(
        _final_kernel,
        grid=(npk // blk3,),
        in_specs=[
            pl.BlockSpec((2, blk3, 128), lambda i: (0, i, 0)),
            pl.BlockSpec((1, 128), lambda i: (0, 0)),
        ],
        out_specs=pl.BlockSpec((blk3, 128), lambda i: (i, 0)),
        out_shape=jax.ShapeDtypeStruct((npk, 128), f32),
    )(acc1v, b1p)

    return outp.reshape(NP, 16)[:N, :7]
